# baseline (speedup 1.0000x reference)
import functools

import jax
import jax.numpy as jnp
from jax import lax
from jax.experimental import pallas as pl
from jax.experimental.pallas import tpu as pltpu
from jax.experimental.pallas import tpu_sc as plsc

NC = 2
NS = 16
L = 16
NW = NC * NS

C_EDGE = 128


def _sc_edge_pass(n_nodes, hid, chunks_per_tile, e_real):
    epw = chunks_per_tile * C_EDGE
    rows_a = ((-(-n_nodes // NS)) + 7) // 8 * 8
    rows_last = n_nodes - (NS - 1) * rows_a
    assert rows_last > 0

    mesh = plsc.VectorSubcoreMesh(core_axis_name="c", subcore_axis_name="s")

    @functools.partial(
        pl.kernel,
        out_type=[
            jax.ShapeDtypeStruct((NC, n_nodes, hid), jnp.float32),
            jax.ShapeDtypeStruct((NC, n_nodes), jnp.float32),
        ],
        mesh=mesh,
        compiler_params=pltpu.CompilerParams(needs_layout_passes=False,
                                             use_tc_tiling_on_sc=False),
        scratch_types=[
            pltpu.VMEM((n_nodes,), jnp.float32),
            pltpu.VMEM((n_nodes,), jnp.float32),
            pltpu.VMEM((chunks_per_tile, C_EDGE), jnp.int32),
            pltpu.VMEM((chunks_per_tile, C_EDGE), jnp.int32),
            pltpu.VMEM((3, C_EDGE), jnp.float32),
            pltpu.VMEM((3, C_EDGE, hid), jnp.float32),
            pltpu.VMEM_SHARED((n_nodes, hid), jnp.float32),
            pltpu.VMEM_SHARED((n_nodes,), jnp.float32),
            [pltpu.SemaphoreType.DMA] * 3,
            [pltpu.SemaphoreType.DMA] * 3,
        ],
    )
    def edge_kernel(h_hbm, asd_hbm, src_hbm, dst_hbm, z2d_hbm, z1d_hbm,
                    num_out, den_out,
                    as_v, ad_v, src_v, dst_v, ea_v, rows_v, num_sh, den_sh,
                    gsem, ssem):
        c = lax.axis_index("c")
        s = lax.axis_index("s")
        wid = c * NS + s

        @pl.when(s < NS - 1)
        def _():
            pltpu.sync_copy(z2d_hbm.at[pl.ds(s * rows_a, rows_a)],
                            num_sh.at[pl.ds(s * rows_a, rows_a)])

        @pl.when(s == NS - 1)
        def _():
            pltpu.sync_copy(z2d_hbm.at[pl.ds((NS - 1) * rows_a, rows_last)],
                            num_sh.at[pl.ds((NS - 1) * rows_a, rows_last)])

        @pl.when(s == 0)
        def _():
            pltpu.sync_copy(z1d_hbm, den_sh)

        pltpu.sync_copy(asd_hbm.at[0], as_v)
        pltpu.sync_copy(asd_hbm.at[1], ad_v)
        pltpu.sync_copy(src_hbm.at[wid], src_v)
        pltpu.sync_copy(dst_hbm.at[wid], dst_v)
        plsc.subcore_barrier()

        base_gid = wid * epw
        assert chunks_per_tile % 3 == 0 and chunks_per_tile >= 6
        n_trips = chunks_per_tile // 3

        pltpu.async_copy(h_hbm.at[src_v.at[0]], rows_v.at[0], gsem[0])
        pltpu.async_copy(h_hbm.at[src_v.at[1]], rows_v.at[1], gsem[1])

        def trip_body(j0, carry):
            for b in range(3):
                j = 3 * j0 + b
                pltpu.make_async_copy(h_hbm.at[src_v.at[j]], rows_v.at[b],
                                      gsem[b]).wait()

                for k in range(C_EDGE // L):
                    srcv = src_v[j, pl.ds(k * L, L)]
                    dstv = dst_v[j, pl.ds(k * L, L)]
                    a_s = plsc.load_gather(as_v, [srcv])
                    a_d = plsc.load_gather(ad_v, [dstv])
                    al = a_s + a_d
                    al = jnp.maximum(al, 0.2 * al)
                    ea = jnp.exp(al)
                    gid = j * C_EDGE + k * L + lax.iota(jnp.int32, L)
                    ea = jnp.where(base_gid + gid < e_real, ea, 0.0)
                    ea_v[b, pl.ds(k * L, L)] = ea

                def scale_body(e, carry2):
                    eb = plsc.load_gather(ea_v.at[b], [jnp.full((L,), e, jnp.int32)])
                    for m in range(hid // L):
                        rows_v[b, e, pl.ds(m * L, L)] = (
                            rows_v[b, e, pl.ds(m * L, L)] * eb)
                    return carry2

                lax.fori_loop(0, C_EDGE, scale_body, 0, unroll=2)

                pltpu.async_copy(rows_v.at[b], num_sh.at[dst_v.at[j]], ssem[b],
                                 add=True)
                pltpu.async_copy(ea_v.at[b], den_sh.at[dst_v.at[j]], ssem[b],
                                 add=True)

                pb = (b + 2) % 3

                def drain():
                    pltpu.make_async_copy(rows_v.at[pb], num_sh.at[dst_v.at[j]],
                                          ssem[pb]).wait()
                    pltpu.make_async_copy(ea_v.at[pb], den_sh.at[dst_v.at[j]],
                                          ssem[pb]).wait()

                if b == 0:
                    @pl.when(j0 >= 1)
                    def _():
                        drain()
                else:
                    drain()

                def fire(jn):
                    pltpu.async_copy(h_hbm.at[src_v.at[jn]], rows_v.at[pb],
                                     gsem[pb])

                if b == 0:
                    fire(j + 2)
                else:
                    @pl.when(j0 < n_trips - 1)
                    def _():
                        fire(j + 2)
            return carry

        lax.fori_loop(0, n_trips, trip_body, 0)
        pltpu.make_async_copy(rows_v.at[2], num_sh.at[dst_v.at[0]],
                              ssem[2]).wait()
        pltpu.make_async_copy(ea_v.at[2], den_sh.at[dst_v.at[0]],
                              ssem[2]).wait()
        plsc.subcore_barrier()

        @pl.when(s < NS - 1)
        def _():
            pltpu.sync_copy(num_sh.at[pl.ds(s * rows_a, rows_a)],
                            num_out.at[c, pl.ds(s * rows_a, rows_a)])

        @pl.when(s == NS - 1)
        def _():
            pltpu.sync_copy(num_sh.at[pl.ds((NS - 1) * rows_a, rows_last)],
                            num_out.at[c, pl.ds((NS - 1) * rows_a, rows_last)])

        @pl.when(s == 0)
        def _():
            pltpu.sync_copy(den_sh, den_out.at[c])

    return edge_kernel


def _tc_first_body(x_ref, w_ref, att_ref, h_ref, asd_ref):
    h = jnp.dot(x_ref[...], w_ref[...], preferred_element_type=jnp.float32)
    h_ref[...] = h
    asd_ref[...] = jnp.dot(h, att_ref[...], preferred_element_type=jnp.float32)


def _tc_mid_body(num_ref, den_ref, bias_ref, scale_ref, beta_ref, w_ref, att_ref,
                 h_ref, asd_ref):
    agg = (num_ref[0] + num_ref[1]) / den_ref[...]
    y = jnp.maximum(agg + bias_ref[...], 0.0)
    xn = y * scale_ref[...] + beta_ref[...]
    h = jnp.dot(xn, w_ref[...], preferred_element_type=jnp.float32)
    h_ref[...] = h
    asd_ref[...] = jnp.dot(h, att_ref[...], preferred_element_type=jnp.float32)


def _tc_final_body(num_ref, den_ref, bias_ref, scale_ref, beta_ref,
                   lw_ref, lb_ref, o_ref):
    agg = (num_ref[0] + num_ref[1]) / den_ref[...]
    y = jnp.maximum(agg + bias_ref[...], 0.0)
    xn = y * scale_ref[...] + beta_ref[...]
    g = jnp.mean(xn, axis=0, keepdims=True)
    o_ref[...] = jnp.dot(g, lw_ref[...], preferred_element_type=jnp.float32) + lb_ref[...]


def kernel(x, edge_index, params):
    n_nodes, d_in = x.shape
    e_edges = edge_index.shape[1]
    hid = params["convs"][0]["W"].shape[1]
    t_out = params["lin_W"].shape[1]

    loops = jnp.arange(n_nodes, dtype=edge_index.dtype)
    src = jnp.concatenate([edge_index[0], loops])
    dst = jnp.concatenate([edge_index[1], loops])
    order = jnp.argsort(dst).astype(jnp.int32)
    src = src[order]
    dst = dst[order]
    e_real = e_edges + n_nodes
    chunks_per_tile = -(-e_real // (NW * C_EDGE))
    e_pad = NW * chunks_per_tile * C_EDGE
    src_p = jnp.pad(src, (0, e_pad - e_real)).reshape(NW, chunks_per_tile, C_EDGE)
    dst_p = jnp.pad(dst, (0, e_pad - e_real)).reshape(NW, chunks_per_tile, C_EDGE)
    z2d = jnp.zeros((n_nodes, hid), jnp.float32)
    z1d = jnp.zeros((n_nodes,), jnp.float32)

    bn_rescale = 1.0 / jnp.sqrt(jnp.float32(1.0 + 1e-5))

    edge_pass = _sc_edge_pass(n_nodes, hid, chunks_per_tile, e_real)

    tc_first = pl.pallas_call(
        _tc_first_body,
        out_shape=[
            jax.ShapeDtypeStruct((n_nodes, hid), jnp.float32),
            jax.ShapeDtypeStruct((n_nodes, 2), jnp.float32),
        ],
    )
    tc_mid = pl.pallas_call(
        _tc_mid_body,
        out_shape=[
            jax.ShapeDtypeStruct((n_nodes, hid), jnp.float32),
            jax.ShapeDtypeStruct((n_nodes, 2), jnp.float32),
        ],
    )
    tc_final = pl.pallas_call(
        _tc_final_body,
        out_shape=jax.ShapeDtypeStruct((1, t_out), jnp.float32),
    )

    conv0 = params["convs"][0]
    att0 = jnp.stack([conv0["att_src"], conv0["att_dst"]], axis=1)
    h, asd = tc_first(x, conv0["W"], att0)

    for i in range(5):
        num, den = edge_pass(h, asd.T, src_p, dst_p, z2d, z1d)
        den_col = (den[0] + den[1]).reshape(n_nodes, 1)
        conv = params["convs"][i]
        bn = params["bns"][i]
        scale = bn["gamma"] * bn_rescale
        if i < 4:
            nconv = params["convs"][i + 1]
            natt = jnp.stack([nconv["att_src"], nconv["att_dst"]], axis=1)
            h, asd = tc_mid(num, den_col, conv["bias"], scale, bn["beta"],
                            nconv["W"], natt)
        else:
            out = tc_final(num, den_col, conv["bias"], scale, bn["beta"],
                           params["lin_W"], params["lin_b"])
    return out


# speedup vs baseline: 1.1601x; 1.1601x over previous
import functools

import jax
import jax.numpy as jnp
from jax import lax
from jax.experimental import pallas as pl
from jax.experimental.pallas import tpu as pltpu
from jax.experimental.pallas import tpu_sc as plsc

NC = 2
NS = 16
L = 16
NW = NC * NS

C_EDGE = 128


def _sc_edge_pass(n_nodes, hid, chunks_per_tile, e_real):
    epw = chunks_per_tile * C_EDGE
    rows_a = ((-(-n_nodes // NS)) + 7) // 8 * 8
    rows_last = n_nodes - (NS - 1) * rows_a
    assert rows_last > 0 and rows_last % 8 == 0
    assert n_nodes % L == 0 and hid % L == 0
    assert chunks_per_tile % 3 == 0 and chunks_per_tile >= 6

    mesh = plsc.VectorSubcoreMesh(core_axis_name="c", subcore_axis_name="s")

    @functools.partial(
        pl.kernel,
        out_type=[
            jax.ShapeDtypeStruct((NC, n_nodes, hid), jnp.float32),
            jax.ShapeDtypeStruct((NC, NS, n_nodes), jnp.float32),
        ],
        mesh=mesh,
        compiler_params=pltpu.CompilerParams(needs_layout_passes=False,
                                             use_tc_tiling_on_sc=False),
        scratch_types=[
            pltpu.VMEM((n_nodes,), jnp.float32),
            pltpu.VMEM((n_nodes,), jnp.float32),
            pltpu.VMEM((chunks_per_tile, C_EDGE), jnp.int32),
            pltpu.VMEM((chunks_per_tile, C_EDGE), jnp.int32),
            pltpu.VMEM((chunks_per_tile // 3, C_EDGE), jnp.int32),
            pltpu.VMEM((chunks_per_tile // 3, C_EDGE), jnp.int32),
            pltpu.VMEM((3, C_EDGE), jnp.float32),
            pltpu.VMEM((3, C_EDGE, hid), jnp.float32),
            pltpu.VMEM((n_nodes,), jnp.float32),
            pltpu.VMEM((rows_a,), jnp.float32),
            pltpu.VMEM((rows_a,), jnp.float32),
            pltpu.VMEM_SHARED((n_nodes, hid), jnp.float32),
            [pltpu.SemaphoreType.DMA] * 3,
            [pltpu.SemaphoreType.DMA] * 3,
        ],
    )
    def edge_kernel(h_hbm, asd_hbm, src_hbm, dst_hbm, z2d_hbm,
                    num_out, den_sg,
                    as_v, ad_v, src_v, dst_v, osrc_v, odst_v, ea_v, rows_v,
                    den_v, dsl_v, tmp_v, num_sh, gsem, ssem):
        c = lax.axis_index("c")
        s = lax.axis_index("s")
        wid = c * NS + s
        owid = (1 - c) * NS + s

        @pl.when(s < NS - 1)
        def _():
            pltpu.sync_copy(z2d_hbm.at[pl.ds(s * rows_a, rows_a)],
                            num_sh.at[pl.ds(s * rows_a, rows_a)])

        @pl.when(s == NS - 1)
        def _():
            pltpu.sync_copy(z2d_hbm.at[pl.ds((NS - 1) * rows_a, rows_last)],
                            num_sh.at[pl.ds((NS - 1) * rows_a, rows_last)])

        pltpu.sync_copy(asd_hbm.at[0], as_v)
        pltpu.sync_copy(asd_hbm.at[1], ad_v)
        pltpu.sync_copy(src_hbm.at[wid], src_v)
        pltpu.sync_copy(dst_hbm.at[wid], dst_v)

        def zden(i, carry):
            den_v[pl.ds(i * L, L)] = jnp.zeros((L,), jnp.float32)
            return carry

        lax.fori_loop(0, n_nodes // L, zden, 0, unroll=8)

        def ea_of(srcv, dstv, gid):
            a_s = plsc.load_gather(as_v, [srcv])
            a_d = plsc.load_gather(ad_v, [dstv])
            al = a_s + a_d
            al = jnp.maximum(al, 0.2 * al)
            ea = jnp.exp(al)
            return jnp.where(gid < e_real, ea, 0.0)

        obase = owid * epw
        sec_chunks = chunks_per_tile // 3
        for sec in range(3):
            pltpu.sync_copy(
                src_hbm.at[owid, pl.ds(sec * sec_chunks, sec_chunks)], osrc_v)
            pltpu.sync_copy(
                dst_hbm.at[owid, pl.ds(sec * sec_chunks, sec_chunks)], odst_v)

            def oden_body(j, carry):
                for k in range(C_EDGE // L):
                    srcv = osrc_v[j, pl.ds(k * L, L)]
                    dstv = odst_v[j, pl.ds(k * L, L)]
                    gid = (obase + (sec * sec_chunks + j) * C_EDGE + k * L
                           + lax.iota(jnp.int32, L))
                    ea = ea_of(srcv, dstv, gid)
                    plsc.addupdate_scatter(den_v, [dstv], ea)
                return carry

            lax.fori_loop(0, sec_chunks, oden_body, 0)

        plsc.subcore_barrier()

        base_gid = wid * epw
        n_trips = chunks_per_tile // 3

        pltpu.async_copy(h_hbm.at[src_v.at[0]], rows_v.at[0], gsem[0])
        pltpu.async_copy(h_hbm.at[src_v.at[1]], rows_v.at[1], gsem[1])

        def trip_body(j0, carry):
            for b in range(3):
                j = 3 * j0 + b
                pltpu.make_async_copy(h_hbm.at[src_v.at[j]], rows_v.at[b],
                                      gsem[b]).wait()

                for k in range(C_EDGE // L):
                    srcv = src_v[j, pl.ds(k * L, L)]
                    dstv = dst_v[j, pl.ds(k * L, L)]
                    gid = base_gid + j * C_EDGE + k * L + lax.iota(jnp.int32, L)
                    ea = ea_of(srcv, dstv, gid)
                    plsc.addupdate_scatter(den_v, [dstv], ea)
                    ea_v[b, pl.ds(k * L, L)] = ea

                def scale_body(e0, carry2):
                    for ei in range(L):
                        e = e0 * L + ei
                        eb = plsc.load_gather(ea_v.at[b],
                                              [jnp.full((L,), e, jnp.int32)])
                        for m in range(hid // L):
                            rows_v[b, e, pl.ds(m * L, L)] = (
                                rows_v[b, e, pl.ds(m * L, L)] * eb)
                    return carry2

                lax.fori_loop(0, C_EDGE // L, scale_body, 0)

                pltpu.async_copy(rows_v.at[b], num_sh.at[dst_v.at[j]], ssem[b],
                                 add=True)

                pb = (b + 2) % 3

                def drain():
                    pltpu.make_async_copy(rows_v.at[pb], num_sh.at[dst_v.at[j]],
                                          ssem[pb]).wait()

                if b == 0:
                    @pl.when(j0 >= 1)
                    def _():
                        drain()
                else:
                    drain()

                def fire(jn):
                    pltpu.async_copy(h_hbm.at[src_v.at[jn]], rows_v.at[pb],
                                     gsem[pb])

                if b == 0:
                    fire(j + 2)
                else:
                    @pl.when(j0 < n_trips - 1)
                    def _():
                        fire(j + 2)
            return carry

        lax.fori_loop(0, n_trips, trip_body, 0)
        pltpu.make_async_copy(rows_v.at[2], num_sh.at[dst_v.at[0]],
                              ssem[2]).wait()

        pltpu.sync_copy(den_v, den_sg.at[c, s])
        plsc.subcore_barrier()

        def normalize_publish(row0, nrows):
            pltpu.sync_copy(den_sg.at[c, 0, pl.ds(row0, nrows)],
                            dsl_v.at[pl.ds(0, nrows)])
            for t in range(1, NS):
                pltpu.sync_copy(den_sg.at[c, t, pl.ds(row0, nrows)],
                                tmp_v.at[pl.ds(0, nrows)])

                def acc(i, carry):
                    dsl_v[pl.ds(i * L, L)] = (dsl_v[pl.ds(i * L, L)]
                                              + tmp_v[pl.ds(i * L, L)])
                    return carry

                lax.fori_loop(0, nrows // L, acc, 0, unroll=4)

            def recip(i, carry):
                dsl_v[pl.ds(i * L, L)] = 1.0 / dsl_v[pl.ds(i * L, L)]
                return carry

            lax.fori_loop(0, nrows // L, recip, 0, unroll=4)

            for p0 in range(0, nrows, C_EDGE):
                pcnt = min(C_EDGE, nrows - p0)
                pltpu.sync_copy(num_sh.at[pl.ds(row0 + p0, pcnt)],
                                rows_v.at[0, pl.ds(0, pcnt)])

                def nrow(r0, carry):
                    for ri in range(8):
                        r = r0 * 8 + ri
                        rv = plsc.load_gather(
                            dsl_v, [jnp.full((L,), p0 + r, jnp.int32)])
                        for m in range(hid // L):
                            rows_v[0, r, pl.ds(m * L, L)] = (
                                rows_v[0, r, pl.ds(m * L, L)] * rv)
                    return carry

                lax.fori_loop(0, pcnt // 8, nrow, 0)
                pltpu.sync_copy(rows_v.at[0, pl.ds(0, pcnt)],
                                num_out.at[c, pl.ds(row0 + p0, pcnt)])

        @pl.when(s < NS - 1)
        def _():
            normalize_publish(s * rows_a, rows_a)

        @pl.when(s == NS - 1)
        def _():
            normalize_publish((NS - 1) * rows_a, rows_last)

    return edge_kernel


def _tc_first_body(x_ref, w_ref, att_ref, h_ref, asd_ref):
    h = jnp.dot(x_ref[...], w_ref[...], preferred_element_type=jnp.float32)
    h_ref[...] = h
    asd_ref[...] = jnp.dot(h, att_ref[...], preferred_element_type=jnp.float32)


def _tc_mid_body(num_ref, bias_ref, scale_ref, beta_ref, w_ref, att_ref,
                 h_ref, asd_ref):
    agg = num_ref[0] + num_ref[1]
    y = jnp.maximum(agg + bias_ref[...], 0.0)
    xn = y * scale_ref[...] + beta_ref[...]
    h = jnp.dot(xn, w_ref[...], preferred_element_type=jnp.float32)
    h_ref[...] = h
    asd_ref[...] = jnp.dot(h, att_ref[...], preferred_element_type=jnp.float32)


def _tc_final_body(num_ref, bias_ref, scale_ref, beta_ref,
                   lw_ref, lb_ref, o_ref):
    agg = num_ref[0] + num_ref[1]
    y = jnp.maximum(agg + bias_ref[...], 0.0)
    xn = y * scale_ref[...] + beta_ref[...]
    g = jnp.mean(xn, axis=0, keepdims=True)
    o_ref[...] = jnp.dot(g, lw_ref[...], preferred_element_type=jnp.float32) + lb_ref[...]


def kernel(x, edge_index, params):
    n_nodes, d_in = x.shape
    e_edges = edge_index.shape[1]
    hid = params["convs"][0]["W"].shape[1]
    t_out = params["lin_W"].shape[1]

    loops = jnp.arange(n_nodes, dtype=edge_index.dtype)
    src = jnp.concatenate([edge_index[0], loops])
    dst = jnp.concatenate([edge_index[1], loops])
    e_real = e_edges + n_nodes
    chunks_per_tile = -(-e_real // (NW * C_EDGE))
    chunks_per_tile = -(-chunks_per_tile // 3) * 3
    e_pad = NW * chunks_per_tile * C_EDGE
    src_p = jnp.pad(src, (0, e_pad - e_real)).reshape(NW, chunks_per_tile, C_EDGE)
    dst_p = jnp.pad(dst, (0, e_pad - e_real)).reshape(NW, chunks_per_tile, C_EDGE)
    z2d = jnp.zeros((n_nodes, hid), jnp.float32)

    bn_rescale = 1.0 / jnp.sqrt(jnp.float32(1.0 + 1e-5))

    edge_pass = _sc_edge_pass(n_nodes, hid, chunks_per_tile, e_real)

    tc_first = pl.pallas_call(
        _tc_first_body,
        out_shape=[
            jax.ShapeDtypeStruct((n_nodes, hid), jnp.float32),
            jax.ShapeDtypeStruct((n_nodes, 2), jnp.float32),
        ],
    )
    tc_mid = pl.pallas_call(
        _tc_mid_body,
        out_shape=[
            jax.ShapeDtypeStruct((n_nodes, hid), jnp.float32),
            jax.ShapeDtypeStruct((n_nodes, 2), jnp.float32),
        ],
    )
    tc_final = pl.pallas_call(
        _tc_final_body,
        out_shape=jax.ShapeDtypeStruct((1, t_out), jnp.float32),
    )

    conv0 = params["convs"][0]
    att0 = jnp.stack([conv0["att_src"], conv0["att_dst"]], axis=1)
    h, asd = tc_first(x, conv0["W"], att0)

    for i in range(5):
        num, _ = edge_pass(h, asd.T, src_p, dst_p, z2d)
        conv = params["convs"][i]
        bn = params["bns"][i]
        scale = bn["gamma"] * bn_rescale
        if i < 4:
            nconv = params["convs"][i + 1]
            natt = jnp.stack([nconv["att_src"], nconv["att_dst"]], axis=1)
            h, asd = tc_mid(num, conv["bias"], scale, bn["beta"],
                            nconv["W"], natt)
        else:
            out = tc_final(num, conv["bias"], scale, bn["beta"],
                           params["lin_W"], params["lin_b"])
    return out


# speedup vs baseline: 1.5306x; 1.3194x over previous
import functools

import jax
import jax.numpy as jnp
from jax import lax
from jax.experimental import pallas as pl
from jax.experimental.pallas import tpu as pltpu
from jax.experimental.pallas import tpu_sc as plsc

NC = 2
NS = 16
L = 16
NW = NC * NS

C_EDGE = 128


def _sc_edge_pass(n_nodes, hid, chunks_per_tile, e_real):
    epw = chunks_per_tile * C_EDGE
    rows_a = ((-(-n_nodes // NS)) + 7) // 8 * 8
    rows_last = n_nodes - (NS - 1) * rows_a
    assert rows_last > 0 and rows_last % 8 == 0
    assert n_nodes % L == 0 and hid % L == 0
    assert chunks_per_tile % 3 == 0 and chunks_per_tile >= 6

    mesh = plsc.VectorSubcoreMesh(core_axis_name="c", subcore_axis_name="s")

    @functools.partial(
        pl.kernel,
        out_type=[
            jax.ShapeDtypeStruct((NC, n_nodes, hid), jnp.float32),
            jax.ShapeDtypeStruct((NC, n_nodes), jnp.float32),
        ],
        mesh=mesh,
        compiler_params=pltpu.CompilerParams(needs_layout_passes=False,
                                             use_tc_tiling_on_sc=False),
        scratch_types=[
            pltpu.VMEM((n_nodes,), jnp.float32),
            pltpu.VMEM((n_nodes,), jnp.float32),
            pltpu.VMEM((chunks_per_tile, C_EDGE), jnp.int32),
            pltpu.VMEM((chunks_per_tile, C_EDGE), jnp.int32),
            pltpu.VMEM((3, C_EDGE), jnp.float32),
            pltpu.VMEM((3, C_EDGE, hid), jnp.float32),
            pltpu.VMEM_SHARED((n_nodes, hid), jnp.float32),
            pltpu.VMEM_SHARED((n_nodes,), jnp.float32),
            [pltpu.SemaphoreType.DMA] * 3,
            [pltpu.SemaphoreType.DMA] * 3,
        ],
    )
    def edge_kernel(h_hbm, asd_hbm, src_hbm, dst_hbm, z2d_hbm, z1d_hbm,
                    num_out, den_out,
                    as_v, ad_v, src_v, dst_v, ea_v, rows_v, num_sh, den_sh,
                    gsem, ssem):
        c = lax.axis_index("c")
        s = lax.axis_index("s")
        wid = c * NS + s

        @pl.when(s < NS - 1)
        def _():
            pltpu.sync_copy(z2d_hbm.at[pl.ds(s * rows_a, rows_a)],
                            num_sh.at[pl.ds(s * rows_a, rows_a)])

        @pl.when(s == NS - 1)
        def _():
            pltpu.sync_copy(z2d_hbm.at[pl.ds((NS - 1) * rows_a, rows_last)],
                            num_sh.at[pl.ds((NS - 1) * rows_a, rows_last)])

        @pl.when(s == 0)
        def _():
            pltpu.sync_copy(z1d_hbm, den_sh)

        pltpu.sync_copy(asd_hbm.at[0], as_v)
        pltpu.sync_copy(asd_hbm.at[1], ad_v)
        pltpu.sync_copy(src_hbm.at[wid], src_v)
        pltpu.sync_copy(dst_hbm.at[wid], dst_v)
        plsc.subcore_barrier()

        base_gid = wid * epw
        n_trips = chunks_per_tile // 3

        pltpu.async_copy(h_hbm.at[src_v.at[0]], rows_v.at[0], gsem[0])
        pltpu.async_copy(h_hbm.at[src_v.at[1]], rows_v.at[1], gsem[1])

        def trip_body(j0, carry):
            for b in range(3):
                j = 3 * j0 + b
                pltpu.make_async_copy(h_hbm.at[src_v.at[j]], rows_v.at[b],
                                      gsem[b]).wait()

                for k in range(C_EDGE // L):
                    srcv = src_v[j, pl.ds(k * L, L)]
                    dstv = dst_v[j, pl.ds(k * L, L)]
                    a_s = plsc.load_gather(as_v, [srcv])
                    a_d = plsc.load_gather(ad_v, [dstv])
                    al = a_s + a_d
                    al = jnp.maximum(al, 0.2 * al)
                    ea = jnp.exp(al)
                    gid = base_gid + j * C_EDGE + k * L + lax.iota(jnp.int32, L)
                    ea = jnp.where(gid < e_real, ea, 0.0)
                    ea_v[b, pl.ds(k * L, L)] = ea

                def scale_body(e, carry2):
                    eb = plsc.load_gather(ea_v.at[b], [jnp.full((L,), e, jnp.int32)])
                    for m in range(hid // L):
                        rows_v[b, e, pl.ds(m * L, L)] = (
                            rows_v[b, e, pl.ds(m * L, L)] * eb)
                    return carry2

                lax.fori_loop(0, C_EDGE, scale_body, 0, unroll=2)

                pltpu.async_copy(rows_v.at[b], num_sh.at[dst_v.at[j]], ssem[b],
                                 add=True)
                pltpu.async_copy(ea_v.at[b], den_sh.at[dst_v.at[j]], ssem[b],
                                 add=True)

                pb = (b + 2) % 3

                def drain():
                    pltpu.make_async_copy(rows_v.at[pb], num_sh.at[dst_v.at[j]],
                                          ssem[pb]).wait()
                    pltpu.make_async_copy(ea_v.at[pb], den_sh.at[dst_v.at[j]],
                                          ssem[pb]).wait()

                if b == 0:
                    @pl.when(j0 >= 1)
                    def _():
                        drain()
                else:
                    drain()

                def fire(jn):
                    pltpu.async_copy(h_hbm.at[src_v.at[jn]], rows_v.at[pb],
                                     gsem[pb])

                if b == 0:
                    fire(j + 2)
                else:
                    @pl.when(j0 < n_trips - 1)
                    def _():
                        fire(j + 2)
            return carry

        lax.fori_loop(0, n_trips, trip_body, 0)
        pltpu.make_async_copy(rows_v.at[2], num_sh.at[dst_v.at[0]],
                              ssem[2]).wait()
        pltpu.make_async_copy(ea_v.at[2], den_sh.at[dst_v.at[0]],
                              ssem[2]).wait()
        plsc.subcore_barrier()

        @pl.when(s < NS - 1)
        def _():
            pltpu.sync_copy(num_sh.at[pl.ds(s * rows_a, rows_a)],
                            num_out.at[c, pl.ds(s * rows_a, rows_a)])

        @pl.when(s == NS - 1)
        def _():
            pltpu.sync_copy(num_sh.at[pl.ds((NS - 1) * rows_a, rows_last)],
                            num_out.at[c, pl.ds((NS - 1) * rows_a, rows_last)])

        @pl.when(s == 0)
        def _():
            pltpu.sync_copy(den_sh, den_out.at[c])

    return edge_kernel


def _tc_first_body(x_ref, w_ref, att_ref, h_ref, asd_ref):
    h = jnp.dot(x_ref[...], w_ref[...], preferred_element_type=jnp.float32)
    h_ref[...] = h
    asd_ref[...] = jnp.dot(h, att_ref[...], preferred_element_type=jnp.float32)


def _tc_mid_body(num_ref, den_ref, bias_ref, scale_ref, beta_ref, w_ref, att_ref,
                 h_ref, asd_ref):
    agg = (num_ref[0] + num_ref[1]) / den_ref[...]
    y = jnp.maximum(agg + bias_ref[...], 0.0)
    xn = y * scale_ref[...] + beta_ref[...]
    h = jnp.dot(xn, w_ref[...], preferred_element_type=jnp.float32)
    h_ref[...] = h
    asd_ref[...] = jnp.dot(h, att_ref[...], preferred_element_type=jnp.float32)


def _tc_final_body(num_ref, den_ref, bias_ref, scale_ref, beta_ref,
                   lw_ref, lb_ref, o_ref):
    agg = (num_ref[0] + num_ref[1]) / den_ref[...]
    y = jnp.maximum(agg + bias_ref[...], 0.0)
    xn = y * scale_ref[...] + beta_ref[...]
    g = jnp.mean(xn, axis=0, keepdims=True)
    o_ref[...] = jnp.dot(g, lw_ref[...], preferred_element_type=jnp.float32) + lb_ref[...]


def kernel(x, edge_index, params):
    n_nodes, d_in = x.shape
    e_edges = edge_index.shape[1]
    hid = params["convs"][0]["W"].shape[1]
    t_out = params["lin_W"].shape[1]

    loops = jnp.arange(n_nodes, dtype=edge_index.dtype)
    src = jnp.concatenate([edge_index[0], loops])
    dst = jnp.concatenate([edge_index[1], loops])
    e_real = e_edges + n_nodes
    chunks_per_tile = -(-e_real // (NW * C_EDGE))
    chunks_per_tile = -(-chunks_per_tile // 3) * 3
    e_pad = NW * chunks_per_tile * C_EDGE
    src_p = jnp.pad(src, (0, e_pad - e_real)).reshape(NW, chunks_per_tile, C_EDGE)
    dst_p = jnp.pad(dst, (0, e_pad - e_real)).reshape(NW, chunks_per_tile, C_EDGE)
    z2d = jnp.zeros((n_nodes, hid), jnp.float32)
    z1d = jnp.zeros((n_nodes,), jnp.float32)

    bn_rescale = 1.0 / jnp.sqrt(jnp.float32(1.0 + 1e-5))

    edge_pass = _sc_edge_pass(n_nodes, hid, chunks_per_tile, e_real)

    tc_first = pl.pallas_call(
        _tc_first_body,
        out_shape=[
            jax.ShapeDtypeStruct((n_nodes, hid), jnp.float32),
            jax.ShapeDtypeStruct((n_nodes, 2), jnp.float32),
        ],
    )
    tc_mid = pl.pallas_call(
        _tc_mid_body,
        out_shape=[
            jax.ShapeDtypeStruct((n_nodes, hid), jnp.float32),
            jax.ShapeDtypeStruct((n_nodes, 2), jnp.float32),
        ],
    )
    tc_final = pl.pallas_call(
        _tc_final_body,
        out_shape=jax.ShapeDtypeStruct((1, t_out), jnp.float32),
    )

    conv0 = params["convs"][0]
    att0 = jnp.stack([conv0["att_src"], conv0["att_dst"]], axis=1)
    h, asd = tc_first(x, conv0["W"], att0)

    for i in range(5):
        num, den = edge_pass(h, asd.T, src_p, dst_p, z2d, z1d)
        den_col = (den[0] + den[1]).reshape(n_nodes, 1)
        conv = params["convs"][i]
        bn = params["bns"][i]
        scale = bn["gamma"] * bn_rescale
        if i < 4:
            nconv = params["convs"][i + 1]
            natt = jnp.stack([nconv["att_src"], nconv["att_dst"]], axis=1)
            h, asd = tc_mid(num, den_col, conv["bias"], scale, bn["beta"],
                            nconv["W"], natt)
        else:
            out = tc_final(num, den_col, conv["bias"], scale, bn["beta"],
                           params["lin_W"], params["lin_b"])
    return out
